# baseline (speedup 1.0000x reference)
import jax
import jax.numpy as jnp
from jax import lax
from jax.experimental import pallas as pl
from jax.experimental.pallas import tpu as pltpu
from jax.experimental.pallas import tpu_sc as plsc

D_MODEL = 2048
N_ROWS = 4 * 2048
NUM_CORES = 2
NUM_SUBCORES = 16
NW = NUM_CORES * NUM_SUBCORES
B_PER_W = N_ROWS // NW
CH = 8
N_CHUNKS = B_PER_W // CH
NSLOT = 3
N_GROUPS = N_CHUNKS // NSLOT
N_PEEL = N_CHUNKS - N_GROUPS * NSLOT


def _pe_add_body(x_hbm, t_hbm, pe_hbm, out_hbm, idx_v,
                 bo0, bo1, bo2, bp0, bp1, bp2,
                 sx0, sx1, sx2, sp0, sp1, sp2, so0, so1, so2):
    bo = (bo0, bo1, bo2)
    bp = (bp0, bp1, bp2)
    sx = (sx0, sx1, sx2)
    sp = (sp0, sp1, sp2)
    so = (so0, so1, so2)

    c = lax.axis_index("c")
    s = lax.axis_index("s")
    wid = s * NUM_CORES + c
    base = wid * B_PER_W
    pltpu.sync_copy(t_hbm.at[pl.ds(base, B_PER_W)], idx_v)

    def start_loads(g, slot):
        row0 = base + g * CH
        pltpu.async_copy(x_hbm.at[pl.ds(row0, CH)], bo[slot], sx[slot])
        pltpu.async_copy(
            pe_hbm.at[idx_v.at[pl.ds(g * CH, CH)]], bp[slot], sp[slot])

    def wait_store(slot):
        pltpu.make_async_copy(bo[slot], out_hbm.at[pl.ds(0, CH)],
                              so[slot]).wait()

    def do_chunk(g, slot, nslot, prefetch, store_pending):
        if prefetch:
            if store_pending:
                wait_store(nslot)
            start_loads(g + 1, nslot)
        pltpu.make_async_copy(
            x_hbm.at[pl.ds(0, CH)], bo[slot], sx[slot]).wait()
        pltpu.make_async_copy(
            pe_hbm.at[pl.ds(0, CH)], bp[slot], sp[slot]).wait()

        def row_add(r, c2):
            for k in range(D_MODEL // 16):
                sl = pl.ds(k * 16, 16)
                plsc.addupdate(bo[slot].at[r, sl], bp[slot][r, sl])
            return c2

        lax.fori_loop(0, CH, row_add, 0)
        row0 = base + g * CH
        pltpu.async_copy(bo[slot], out_hbm.at[pl.ds(row0, CH)], so[slot])

    start_loads(0, 0)

    do_chunk(0, 0, 1, True, False)
    do_chunk(1, 1, 2, True, False)

    def group(gg, carry):
        g0 = 2 + gg * NSLOT
        do_chunk(g0 + 0, 2, 0, True, True)
        do_chunk(g0 + 1, 0, 1, True, True)
        do_chunk(g0 + 2, 1, 2, True, True)
        return carry

    lax.fori_loop(0, (N_CHUNKS - 2 - NSLOT) // NSLOT, group, 0)

    g_tail = N_CHUNKS - NSLOT
    do_chunk(g_tail + 0, 2, 0, True, True)
    do_chunk(g_tail + 1, 0, 1, True, True)
    do_chunk(g_tail + 2, 1, 2, False, False)

    for slot in range(NSLOT):
        wait_store(slot)


def kernel(x, t, pe):
    b, s, d = x.shape
    x2 = x.reshape(N_ROWS, D_MODEL)
    t1 = t.reshape(N_ROWS)

    mesh = plsc.VectorSubcoreMesh(
        core_axis_name="c",
        subcore_axis_name="s",
        num_cores=NUM_CORES,
        num_subcores=NUM_SUBCORES,
    )
    buf = pltpu.VMEM((CH, D_MODEL), jnp.float32)
    sem = pltpu.SemaphoreType.DMA
    run = pl.kernel(
        _pe_add_body,
        out_type=jax.ShapeDtypeStruct((N_ROWS, D_MODEL), jnp.float32),
        mesh=mesh,
        scratch_types=[
            pltpu.VMEM((B_PER_W,), jnp.int32),
            buf, buf, buf, buf, buf, buf,
            sem, sem, sem, sem, sem, sem, sem, sem, sem,
        ],
    )
    out = run(x2, t1, pe)
    return out.reshape(b, s, d)
